# baseline (speedup 1.0000x reference)
import math

import jax
import jax.numpy as jnp
import numpy as np
from jax import lax
from jax.experimental import pallas as pl
from jax.experimental.pallas import tpu as pltpu
from jax.experimental.pallas import tpu_sc as plsc

D_MODEL = 128
SEQ = 200
BATCH = 4096
SCALE = math.sqrt(128.0)
LANES = 16
NW = 32
NBATCH_PER_W = BATCH // NW
CHUNK_A = 128
CHUNK_B = SEQ - CHUNK_A


def _positional_encoding(length, depth):
    half = depth / 2
    positions = np.arange(length)[:, np.newaxis]
    depths = np.arange(half)[np.newaxis, :] / half
    angle_rates = 1 / 1000 ** depths
    angle_rads = positions * angle_rates
    return np.concatenate(
        [np.sin(angle_rads), np.cos(angle_rads)], axis=-1
    ).astype(np.float32)


_PE = _positional_encoding(SEQ, D_MODEL)


def _sc_body(x_ref, table_ref, pe_hbm, out_ref,
             pe_v, rows, idx_a, idx_b, sem_a, sem_b):
    c = lax.axis_index("c")
    s = lax.axis_index("s")
    wid = s * 2 + c
    pltpu.sync_copy(pe_hbm, pe_v)

    def batch_body(b, carry):
        base = (wid * NBATCH_PER_W + b) * SEQ
        pltpu.sync_copy(x_ref.at[pl.ds(base, CHUNK_A)], idx_a)
        pltpu.sync_copy(x_ref.at[pl.ds(base + CHUNK_A, CHUNK_B)], idx_b)
        cp_a = pltpu.async_copy(table_ref.at[idx_a],
                                rows.at[pl.ds(0, CHUNK_A)], sem_a)
        cp_b = pltpu.async_copy(table_ref.at[idx_b],
                                rows.at[pl.ds(CHUNK_A, CHUNK_B)], sem_b)
        cp_a.wait()
        cp_b.wait()

        def row_body(r, carry2):
            for v in range(D_MODEL // LANES):
                sl = pl.ds(v * LANES, LANES)
                rows[r, sl] = rows[r, sl] * SCALE + pe_v[r, sl]
            return carry2

        lax.fori_loop(0, SEQ, row_body, 0)
        pltpu.sync_copy(rows, out_ref.at[pl.ds(base, SEQ)])
        return carry

    lax.fori_loop(0, NBATCH_PER_W, batch_body, 0)


@jax.jit
def _impl(x, table):
    xf = x.reshape(-1)
    mesh = plsc.VectorSubcoreMesh(core_axis_name="c", subcore_axis_name="s")
    out = pl.kernel(
        _sc_body,
        out_type=jax.ShapeDtypeStruct((BATCH * SEQ, D_MODEL), jnp.float32),
        mesh=mesh,
        scratch_types=[
            pltpu.VMEM((SEQ, D_MODEL), jnp.float32),
            pltpu.VMEM((SEQ, D_MODEL), jnp.float32),
            pltpu.VMEM((CHUNK_A,), jnp.int32),
            pltpu.VMEM((CHUNK_B,), jnp.int32),
            pltpu.SemaphoreType.DMA,
            pltpu.SemaphoreType.DMA,
        ],
    )(xf, table, jnp.asarray(_PE))
    return out.reshape(BATCH, SEQ, D_MODEL)


def kernel(x, table):
    return _impl(x, table)


# speedup vs baseline: 1.6004x; 1.6004x over previous
import math

import jax
import jax.numpy as jnp
import numpy as np
from jax import lax
from jax.experimental import pallas as pl
from jax.experimental.pallas import tpu as pltpu
from jax.experimental.pallas import tpu_sc as plsc

D_MODEL = 128
SEQ = 200
BATCH = 4096
SCALE = math.sqrt(128.0)
LANES = 16
NW = 32
NBATCH_PER_W = BATCH // NW
CHUNK_A = 128
CHUNK_B = SEQ - CHUNK_A


def _positional_encoding(length, depth):
    half = depth / 2
    positions = np.arange(length)[:, np.newaxis]
    depths = np.arange(half)[np.newaxis, :] / half
    angle_rates = 1 / 1000 ** depths
    angle_rads = positions * angle_rates
    return np.concatenate(
        [np.sin(angle_rads), np.cos(angle_rads)], axis=-1
    ).astype(np.float32)


_PE = _positional_encoding(SEQ, D_MODEL)


def _sc_body(x_ref, table_ref, pe_hbm, out_ref,
             pe_v, r0, r1, ia0, ib0, ia1, ib1,
             g0, g1, s0, s1):
    c = lax.axis_index("c")
    s = lax.axis_index("s")
    wid = s * 2 + c
    batch0 = wid * NBATCH_PER_W
    pltpu.sync_copy(pe_hbm, pe_v)

    def start_gather(b, ia, ib, rows, sem):
        base = b * SEQ
        pltpu.sync_copy(x_ref.at[pl.ds(base, CHUNK_A)], ia)
        pltpu.sync_copy(x_ref.at[pl.ds(base + CHUNK_A, CHUNK_B)], ib)
        pltpu.async_copy(table_ref.at[ia], rows.at[pl.ds(0, CHUNK_A)], sem)
        pltpu.async_copy(table_ref.at[ib],
                         rows.at[pl.ds(CHUNK_A, CHUNK_B)], sem)

    def wait_gather(ia, ib, rows, sem):
        pltpu.make_async_copy(table_ref.at[ia],
                              rows.at[pl.ds(0, CHUNK_A)], sem).wait()
        pltpu.make_async_copy(table_ref.at[ib],
                              rows.at[pl.ds(CHUNK_A, CHUNK_B)], sem).wait()

    def compute(rows):
        def row_body(r, carry):
            for v in range(D_MODEL // LANES):
                sl = pl.ds(v * LANES, LANES)
                rows[r, sl] = rows[r, sl] * SCALE + pe_v[r, sl]
            return carry

        lax.fori_loop(0, SEQ, row_body, 0)

    start_gather(batch0 + 0, ia0, ib0, r0, g0)
    start_gather(batch0 + 1, ia1, ib1, r1, g1)

    def iter_body(k, carry):
        b0 = batch0 + 2 * k
        b1 = b0 + 1
        wait_gather(ia0, ib0, r0, g0)
        compute(r0)
        st0 = pltpu.async_copy(r0, out_ref.at[pl.ds(b0 * SEQ, SEQ)], s0)
        wait_gather(ia1, ib1, r1, g1)
        compute(r1)
        st1 = pltpu.async_copy(r1, out_ref.at[pl.ds(b1 * SEQ, SEQ)], s1)

        @pl.when(k < NBATCH_PER_W // 2 - 1)
        def _refill():
            st0.wait()
            start_gather(b0 + 2, ia0, ib0, r0, g0)
            st1.wait()
            start_gather(b1 + 2, ia1, ib1, r1, g1)

        @pl.when(k == NBATCH_PER_W // 2 - 1)
        def _drain():
            st0.wait()
            st1.wait()

        return carry

    lax.fori_loop(0, NBATCH_PER_W // 2, iter_body, 0)


@jax.jit
def _impl(x, table):
    xf = x.reshape(-1)
    mesh = plsc.VectorSubcoreMesh(core_axis_name="c", subcore_axis_name="s")
    out = pl.kernel(
        _sc_body,
        out_type=jax.ShapeDtypeStruct((BATCH * SEQ, D_MODEL), jnp.float32),
        mesh=mesh,
        scratch_types=[
            pltpu.VMEM((SEQ, D_MODEL), jnp.float32),
            pltpu.VMEM((SEQ, D_MODEL), jnp.float32),
            pltpu.VMEM((SEQ, D_MODEL), jnp.float32),
            pltpu.VMEM((CHUNK_A,), jnp.int32),
            pltpu.VMEM((CHUNK_B,), jnp.int32),
            pltpu.VMEM((CHUNK_A,), jnp.int32),
            pltpu.VMEM((CHUNK_B,), jnp.int32),
            pltpu.SemaphoreType.DMA,
            pltpu.SemaphoreType.DMA,
            pltpu.SemaphoreType.DMA,
            pltpu.SemaphoreType.DMA,
        ],
    )(xf, table, jnp.asarray(_PE))
    return out.reshape(BATCH, SEQ, D_MODEL)


def kernel(x, table):
    return _impl(x, table)


# speedup vs baseline: 1.7641x; 1.1023x over previous
import math

import jax
import jax.numpy as jnp
import numpy as np
from jax import lax
from jax.experimental import pallas as pl
from jax.experimental.pallas import tpu as pltpu
from jax.experimental.pallas import tpu_sc as plsc

D_MODEL = 128
SEQ = 200
BATCH = 4096
SCALE = math.sqrt(128.0)
LANES = 16
NW = 32
NB = BATCH // NW
NBUF = 4
CHUNK_A = 128
CHUNK_B = SEQ - CHUNK_A


def _positional_encoding(length, depth):
    half = depth / 2
    positions = np.arange(length)[:, np.newaxis]
    depths = np.arange(half)[np.newaxis, :] / half
    angle_rates = 1 / 1000 ** depths
    angle_rads = positions * angle_rates
    return np.concatenate(
        [np.sin(angle_rads), np.cos(angle_rads)], axis=-1
    ).astype(np.float32)


_PE = _positional_encoding(SEQ, D_MODEL)


def _sc_body(x_ref, table_ref, pe_hbm, out_ref, pe_v, *scratch):
    rows = scratch[0:NBUF]
    ia = scratch[NBUF:2 * NBUF]
    ib = scratch[2 * NBUF:3 * NBUF]
    g = scratch[3 * NBUF:4 * NBUF]
    st = scratch[4 * NBUF:5 * NBUF]

    c = lax.axis_index("c")
    s = lax.axis_index("s")
    wid = s * 2 + c
    batch0 = wid * NB
    pltpu.sync_copy(pe_hbm, pe_v)

    def start_gather(b, j):
        base = (batch0 + b) * SEQ
        pltpu.sync_copy(x_ref.at[pl.ds(base, CHUNK_A)], ia[j])
        pltpu.sync_copy(x_ref.at[pl.ds(base + CHUNK_A, CHUNK_B)], ib[j])
        pltpu.async_copy(table_ref.at[ia[j]],
                         rows[j].at[pl.ds(0, CHUNK_A)], g[j])
        pltpu.async_copy(table_ref.at[ib[j]],
                         rows[j].at[pl.ds(CHUNK_A, CHUNK_B)], g[j])

    def wait_gather(j):
        pltpu.make_async_copy(table_ref.at[ia[j]],
                              rows[j].at[pl.ds(0, CHUNK_A)], g[j]).wait()
        pltpu.make_async_copy(table_ref.at[ib[j]],
                              rows[j].at[pl.ds(CHUNK_A, CHUNK_B)], g[j]).wait()

    def wait_store(b, j):
        pltpu.make_async_copy(
            rows[j], out_ref.at[pl.ds((batch0 + b) * SEQ, SEQ)], st[j]).wait()

    def compute(j):
        rj = rows[j]

        def row_body(r, carry):
            for v in range(D_MODEL // LANES):
                sl = pl.ds(v * LANES, LANES)
                rj[r, sl] = rj[r, sl] * SCALE + pe_v[r, sl]
            return carry

        lax.fori_loop(0, SEQ, row_body, 0)

    for j in range(NBUF - 1):
        start_gather(j, j)

    K = NB // NBUF

    def iter_body(k, carry):
        for j in range(NBUF):
            b = NBUF * k + j
            wait_gather(j)
            compute(j)
            pltpu.async_copy(
                rows[j], out_ref.at[pl.ds((batch0 + b) * SEQ, SEQ)], st[j])
            jp = (j + NBUF - 1) % NBUF
            if j == 0:
                @pl.when(k > 0)
                def _w():
                    wait_store(NBUF * k - 1, jp)

                start_gather(b + NBUF - 1, jp)
            else:
                @pl.when(k < K - 1)
                def _w2():
                    wait_store(b - 1, jp)
                    start_gather(b + NBUF - 1, jp)
        return carry

    lax.fori_loop(0, K, iter_body, 0)

    for j in range(NBUF):
        wait_store(NB - NBUF + j, j)


@jax.jit
def _impl(x, table):
    xf = x.reshape(-1)
    mesh = plsc.VectorSubcoreMesh(core_axis_name="c", subcore_axis_name="s")
    scratch = (
        [pltpu.VMEM((SEQ, D_MODEL), jnp.float32)]
        + [pltpu.VMEM((SEQ, D_MODEL), jnp.float32)] * NBUF
        + [pltpu.VMEM((CHUNK_A,), jnp.int32)] * NBUF
        + [pltpu.VMEM((CHUNK_B,), jnp.int32)] * NBUF
        + [pltpu.SemaphoreType.DMA] * NBUF
        + [pltpu.SemaphoreType.DMA] * NBUF
    )
    out = pl.kernel(
        _sc_body,
        out_type=jax.ShapeDtypeStruct((BATCH * SEQ, D_MODEL), jnp.float32),
        mesh=mesh,
        scratch_types=scratch,
    )(xf, table, jnp.asarray(_PE))
    return out.reshape(BATCH, SEQ, D_MODEL)


def kernel(x, table):
    return _impl(x, table)


# speedup vs baseline: 2.1972x; 1.2455x over previous
import math

import jax
import jax.numpy as jnp
import numpy as np
from jax import lax
from jax.experimental import pallas as pl
from jax.experimental.pallas import tpu as pltpu
from jax.experimental.pallas import tpu_sc as plsc

D_MODEL = 128
SEQ = 200
BATCH = 4096
SCALE = math.sqrt(128.0)
LANES = 16
NW = 32
NB = BATCH // NW
NBUF = 4
CHUNK_A = 128
CHUNK_B = SEQ - CHUNK_A


def _positional_encoding(length, depth):
    half = depth / 2
    positions = np.arange(length)[:, np.newaxis]
    depths = np.arange(half)[np.newaxis, :] / half
    angle_rates = 1 / 1000 ** depths
    angle_rads = positions * angle_rates
    return np.concatenate(
        [np.sin(angle_rads), np.cos(angle_rads)], axis=-1
    ).astype(np.float32)


_PE = _positional_encoding(SEQ, D_MODEL)


def _sc_body(x_ref, table_ref, pe_hbm, out_ref, pe_v, idx0, idx1, *scratch):
    rows = scratch[0:NBUF]
    g = scratch[NBUF:2 * NBUF]
    st = scratch[2 * NBUF:3 * NBUF]
    idx = (idx0, idx1)

    c = lax.axis_index("c")
    s = lax.axis_index("s")
    wid = s * 2 + c
    batch0 = wid * NB
    pltpu.sync_copy(pe_hbm, pe_v)

    def load_idx_block(k, blk):
        base = (batch0 + NBUF * k) * SEQ
        pltpu.sync_copy(x_ref.at[pl.ds(base, NBUF * SEQ)], idx[blk])

    def start_gather(pos, blk, j):
        pltpu.async_copy(table_ref.at[idx[blk].at[pl.ds(pos * SEQ, CHUNK_A)]],
                         rows[j].at[pl.ds(0, CHUNK_A)], g[j])
        pltpu.async_copy(
            table_ref.at[idx[blk].at[pl.ds(pos * SEQ + CHUNK_A, CHUNK_B)]],
            rows[j].at[pl.ds(CHUNK_A, CHUNK_B)], g[j])

    def wait_gather(j):
        pltpu.make_async_copy(
            table_ref.at[idx0.at[pl.ds(0, CHUNK_A)]],
            rows[j].at[pl.ds(0, CHUNK_A)], g[j]).wait()
        pltpu.make_async_copy(
            table_ref.at[idx0.at[pl.ds(CHUNK_A, CHUNK_B)]],
            rows[j].at[pl.ds(CHUNK_A, CHUNK_B)], g[j]).wait()

    def start_store(b, j):
        pltpu.async_copy(
            rows[j], out_ref.at[pl.ds((batch0 + b) * SEQ, SEQ)], st[j])

    def wait_store(b, j):
        pltpu.make_async_copy(
            rows[j], out_ref.at[pl.ds((batch0 + b) * SEQ, SEQ)], st[j]).wait()

    def compute(j):
        rj = rows[j]

        def row_body(r, carry):
            for v in range(D_MODEL // LANES):
                sl = pl.ds(v * LANES, LANES)
                rj[r, sl] = rj[r, sl] * SCALE + pe_v[r, sl]
            return carry

        lax.fori_loop(0, SEQ, row_body, 0)

    load_idx_block(0, 0)
    start_gather(0, 0, 0)
    start_gather(1, 0, 1)

    K = NB // NBUF
    M = K // 2

    def iter_body(m, carry):
        for kk in range(2):
            k = 2 * m + kk
            blk, nblk = kk, 1 - kk
            for j in range(NBUF):
                b = NBUF * k + j
                if j == 0:
                    if kk == 0:
                        load_idx_block(k + 1, nblk)
                    else:
                        @pl.when(m < M - 1)
                        def _ld():
                            load_idx_block(k + 1, nblk)

                wait_gather(j)
                compute(j)
                start_store(b, j)

                jn = (j + 2) % NBUF
                if j < 2:
                    if kk == 0:
                        @pl.when(m > 0)
                        def _w():
                            wait_store(b - 2, jn)
                    else:
                        wait_store(b - 2, jn)
                    start_gather(j + 2, blk, jn)
                else:
                    if kk == 0:
                        wait_store(b - 2, jn)
                        start_gather(j - 2, nblk, jn)
                    else:
                        @pl.when(m < M - 1)
                        def _w2():
                            wait_store(b - 2, jn)
                            start_gather(j - 2, nblk, jn)
        return carry

    lax.fori_loop(0, M, iter_body, 0)

    for j in range(NBUF):
        wait_store(NB - NBUF + j, j)


@jax.jit
def _impl(x, table):
    xf = x.reshape(-1)
    mesh = plsc.VectorSubcoreMesh(core_axis_name="c", subcore_axis_name="s")
    scratch = (
        [pltpu.VMEM((SEQ, D_MODEL), jnp.float32)]
        + [pltpu.VMEM((NBUF * SEQ,), jnp.int32)] * 2
        + [pltpu.VMEM((SEQ, D_MODEL), jnp.float32)] * NBUF
        + [pltpu.SemaphoreType.DMA] * NBUF
        + [pltpu.SemaphoreType.DMA] * NBUF
    )
    out = pl.kernel(
        _sc_body,
        out_type=jax.ShapeDtypeStruct((BATCH * SEQ, D_MODEL), jnp.float32),
        mesh=mesh,
        scratch_types=scratch,
    )(xf, table, jnp.asarray(_PE))
    return out.reshape(BATCH, SEQ, D_MODEL)


def kernel(x, table):
    return _impl(x, table)
